# baseline (speedup 1.0000x reference)
import functools

import jax
import jax.numpy as jnp
from jax import lax
from jax.experimental import pallas as pl
from jax.experimental.pallas import tpu as pltpu
from jax.experimental.pallas import tpu_sc as plsc

N = 10000
N_PAD = 10240
D = 128
NC, NS, L = 2, 16, 16
BM = 1000
GRID_M = N // BM
CH = 64
NCH = 160
ST = 32
NST = NCH // ST
NG = NC * NS
E_PAD = NG * NCH * CH
ROWS_PER_TILE = N_PAD // NS


def _mm0_body(x_ref, w_ref, p_ref):
    p = jnp.dot(x_ref[...], w_ref[...], preferred_element_type=jnp.float32)
    p_ref[0] = p[:, :D]
    p_ref[1] = p[:, D:]


def _mm0(x, W0):
    return pl.pallas_call(
        _mm0_body,
        grid=(GRID_M,),
        in_specs=[pl.BlockSpec((BM, 128), lambda i: (i, 0)),
                  pl.BlockSpec((128, 256), lambda i: (0, 0))],
        out_specs=pl.BlockSpec((2, BM, D), lambda i: (0, i, 0)),
        out_shape=jax.ShapeDtypeStruct((2, N, D), jnp.float32),
    )(x, W0)


def _act_mm_body(has_prev, emit_h, g_ref, b_ref, *rest):
    if has_prev:
        prev_ref, w_ref = rest[0], rest[1]
        outs = rest[2:]
    else:
        w_ref = rest[0]
        outs = rest[1:]
    h0 = jax.nn.relu(g_ref[0] + b_ref[0])
    h1 = jax.nn.relu(g_ref[1] + b_ref[1])
    if has_prev:
        h0 = h0 + prev_ref[0]
        h1 = h1 + prev_ref[1]
    p = (jnp.dot(h0, w_ref[0], preferred_element_type=jnp.float32)
         + jnp.dot(h1, w_ref[1], preferred_element_type=jnp.float32))
    if emit_h:
        h_ref, p_ref = outs
        h_ref[0] = h0
        h_ref[1] = h1
        p_ref[0] = p[:, :D]
        p_ref[1] = p[:, D:]
    else:
        (p_ref,) = outs
        p_ref[...] = p


def _act_mm(g, b2, prev, Wr, emit_h):
    dout = Wr.shape[2]
    has_prev = prev is not None
    in_specs = [pl.BlockSpec((2, BM, D), lambda i: (0, i, 0)),
                pl.BlockSpec((2, 1, D), lambda i: (0, 0, 0))]
    args = [g, b2]
    if has_prev:
        in_specs.append(pl.BlockSpec((2, BM, D), lambda i: (0, i, 0)))
        args.append(prev)
    in_specs.append(pl.BlockSpec((2, 128, dout), lambda i: (0, 0, 0)))
    args.append(Wr)
    if emit_h:
        out_specs = [pl.BlockSpec((2, BM, D), lambda i: (0, i, 0)),
                     pl.BlockSpec((2, BM, D), lambda i: (0, i, 0))]
        out_shape = [jax.ShapeDtypeStruct((2, N, D), jnp.float32),
                     jax.ShapeDtypeStruct((2, N, D), jnp.float32)]
    else:
        out_specs = pl.BlockSpec((BM, dout), lambda i: (i, 0))
        out_shape = jax.ShapeDtypeStruct((N, dout), jnp.float32)
    return pl.pallas_call(
        functools.partial(_act_mm_body, has_prev, emit_h),
        grid=(GRID_M,),
        in_specs=in_specs,
        out_specs=out_specs,
        out_shape=out_shape,
    )(*args)


def _final_body(parts_ref, b_ref, out_ref):
    out_ref[...] = parts_ref[0] + parts_ref[1] + b_ref[...]


def _final(parts, b3):
    return pl.pallas_call(
        _final_body,
        grid=(GRID_M,),
        in_specs=[pl.BlockSpec((2, BM, D), lambda i: (0, i, 0)),
                  pl.BlockSpec((1, D), lambda i: (0, 0))],
        out_specs=pl.BlockSpec((BM, D), lambda i: (i, 0)),
        out_shape=jax.ShapeDtypeStruct((N, D), jnp.float32),
    )(parts, b3.reshape(1, D))


_GDN = lax.GatherDimensionNumbers(
    offset_dims=(), collapsed_slice_dims=(0,), start_index_map=(0,))


def _lane_splat(v, i):
    idx = jnp.zeros((L, 1), jnp.int32) + i
    return lax.gather(v, idx, _GDN, (1,),
                      mode=lax.GatherScatterMode.PROMISE_IN_BOUNDS)

def _spmm_body(col_split, p_hbm, src_hbm, dst_hbm, w_hbm, out_hbm,
               acc, srcb, dstb, wb, rows0, rows1,
               gsem0, gsem1, ssem0, ssem1):
    c = lax.axis_index("c")
    s = lax.axis_index("s")
    zero16 = jnp.zeros((16,), jnp.float32)
    rows = (rows0, rows1)
    gsem = (gsem0, gsem1)
    ssem = (ssem0, ssem1)

    def zrow(r, _):
        rr = rows0.at[r]
        for q in range(8):
            rr[pl.ds(q * 16, 16)] = zero16
        return 0
    lax.fori_loop(0, CH, zrow, 0)
    base = s * ROWS_PER_TILE
    for k in range(ROWS_PER_TILE // CH):
        pltpu.sync_copy(rows0, acc.at[pl.ds(base + k * CH, CH)])

    if col_split:
        groups = (2 * s, 2 * s + 1)
        table = p_hbm.at[c]
    else:
        groups = (c * NS + s,)
        table = p_hbm.at[0]

    for g in groups:
        for st in range(NST):
            pltpu.sync_copy(src_hbm.at[g, pl.ds(st * ST, ST)], srcb)
            pltpu.sync_copy(dst_hbm.at[g, pl.ds(st * ST, ST)], dstb)
            pltpu.sync_copy(w_hbm.at[g, pl.ds(st * ST, ST)], wb)

            def chunk_body(j, _):
                g_d = pltpu.make_async_copy(table.at[srcb.at[j]], rows0,
                                            gsem0)
                g_d.start()
                g_d.wait()
                wrow = wb.at[j]

                def srow(r, _):
                    rbase = (r // L) * L
                    wv16 = wrow[pl.ds(rbase, L)]
                    wsp = _lane_splat(wv16, r - rbase)
                    rr = rows0.at[r]
                    for q in range(8):
                        sl = pl.ds(q * L, L)
                        rr[sl] = rr[sl] * wsp
                    return 0
                lax.fori_loop(0, CH, srow, 0)
                pltpu.async_copy(rows0, acc.at[dstb.at[j]], ssem0, add=True)
                pltpu.make_async_copy(rows0, acc.at[dstb.at[j]],
                                      ssem0).wait()
                return 0
            lax.fori_loop(0, ST, chunk_body, 0)

    plsc.subcore_barrier()
    pltpu.sync_copy(acc.at[pl.ds(base, ROWS_PER_TILE)],
                    out_hbm.at[c, pl.ds(base, ROWS_PER_TILE)])


def _make_spmm(col_split):
    mesh = plsc.VectorSubcoreMesh(core_axis_name="c", subcore_axis_name="s",
                                  num_cores=NC, num_subcores=NS)
    return pl.kernel(
        functools.partial(_spmm_body, col_split),
        out_type=jax.ShapeDtypeStruct((2, N_PAD, D), jnp.float32),
        mesh=mesh,
        scratch_types=[
            pltpu.VMEM_SHARED((N_PAD, D), jnp.float32),
            pltpu.VMEM((ST, CH), jnp.int32),
            pltpu.VMEM((ST, CH), jnp.int32),
            pltpu.VMEM((ST, CH), jnp.float32),
            pltpu.VMEM((CH, D), jnp.float32),
            pltpu.VMEM((CH, D), jnp.float32),
            pltpu.SemaphoreType.DMA,
            pltpu.SemaphoreType.DMA,
            pltpu.SemaphoreType.DMA,
            pltpu.SemaphoreType.DMA,
        ],
    )


def kernel(x, edge_index, edge_weight, W0, b0, W1, b1, W2, b2, W3, b3):
    E = edge_index.shape[1]
    pad = E_PAD - E
    src3 = jnp.pad(edge_index[0], (0, pad)).reshape(NG, NCH, CH)
    dst3 = jnp.pad(edge_index[1], (0, pad)).reshape(NG, NCH, CH)
    w3 = jnp.pad(edge_weight, (0, pad)).reshape(NG, NCH, CH)

    spmm256 = _make_spmm(True)
    spmm128 = _make_spmm(False)

    p = _mm0(x, W0)
    g0 = spmm256(p, src3, dst3, w3)
    h0, p1 = _act_mm(g0, b0.reshape(2, 1, D), None,
                     W1.reshape(2, 128, 256), True)
    g1 = spmm256(p1, src3, dst3, w3)
    h1, p2 = _act_mm(g1, b1.reshape(2, 1, D), h0,
                     W2.reshape(2, 128, 256), True)
    g2 = spmm256(p2, src3, dst3, w3)
    p3 = _act_mm(g2, b2.reshape(2, 1, D), h1,
                 W3.reshape(2, 128, 128), False)
    parts = spmm128(p3.reshape(1, N, D), src3, dst3, w3)
    return _final(parts, b3)


# speedup vs baseline: 1.1402x; 1.1402x over previous
import functools

import jax
import jax.numpy as jnp
from jax import lax
from jax.experimental import pallas as pl
from jax.experimental.pallas import tpu as pltpu
from jax.experimental.pallas import tpu_sc as plsc

N = 10000
N_PAD = 10240
D = 128
NC, NS, L = 2, 16, 16
BM = 1000
GRID_M = N // BM
CH = 64
NCH = 160
ST = 32
NST = NCH // ST
NG = NC * NS
E_PAD = NG * NCH * CH
ROWS_PER_TILE = N_PAD // NS


def _mm0_body(x_ref, w_ref, p_ref):
    p = jnp.dot(x_ref[...], w_ref[...], preferred_element_type=jnp.float32)
    p_ref[0] = p[:, :D]
    p_ref[1] = p[:, D:]


def _mm0(x, W0):
    return pl.pallas_call(
        _mm0_body,
        grid=(GRID_M,),
        in_specs=[pl.BlockSpec((BM, 128), lambda i: (i, 0)),
                  pl.BlockSpec((128, 256), lambda i: (0, 0))],
        out_specs=pl.BlockSpec((2, BM, D), lambda i: (0, i, 0)),
        out_shape=jax.ShapeDtypeStruct((2, N, D), jnp.float32),
    )(x, W0)


def _act_mm_body(has_prev, emit_h, g_ref, b_ref, *rest):
    if has_prev:
        prev_ref, w_ref = rest[0], rest[1]
        outs = rest[2:]
    else:
        w_ref = rest[0]
        outs = rest[1:]
    h0 = jax.nn.relu(g_ref[0] + b_ref[0])
    h1 = jax.nn.relu(g_ref[1] + b_ref[1])
    if has_prev:
        h0 = h0 + prev_ref[0]
        h1 = h1 + prev_ref[1]
    p = (jnp.dot(h0, w_ref[0], preferred_element_type=jnp.float32)
         + jnp.dot(h1, w_ref[1], preferred_element_type=jnp.float32))
    if emit_h:
        h_ref, p_ref = outs
        h_ref[0] = h0
        h_ref[1] = h1
        p_ref[0] = p[:, :D]
        p_ref[1] = p[:, D:]
    else:
        (p_ref,) = outs
        p_ref[...] = p


def _act_mm(g, b2, prev, Wr, emit_h):
    dout = Wr.shape[2]
    has_prev = prev is not None
    in_specs = [pl.BlockSpec((2, BM, D), lambda i: (0, i, 0)),
                pl.BlockSpec((2, 1, D), lambda i: (0, 0, 0))]
    args = [g, b2]
    if has_prev:
        in_specs.append(pl.BlockSpec((2, BM, D), lambda i: (0, i, 0)))
        args.append(prev)
    in_specs.append(pl.BlockSpec((2, 128, dout), lambda i: (0, 0, 0)))
    args.append(Wr)
    if emit_h:
        out_specs = [pl.BlockSpec((2, BM, D), lambda i: (0, i, 0)),
                     pl.BlockSpec((2, BM, D), lambda i: (0, i, 0))]
        out_shape = [jax.ShapeDtypeStruct((2, N, D), jnp.float32),
                     jax.ShapeDtypeStruct((2, N, D), jnp.float32)]
    else:
        out_specs = pl.BlockSpec((BM, dout), lambda i: (i, 0))
        out_shape = jax.ShapeDtypeStruct((N, dout), jnp.float32)
    return pl.pallas_call(
        functools.partial(_act_mm_body, has_prev, emit_h),
        grid=(GRID_M,),
        in_specs=in_specs,
        out_specs=out_specs,
        out_shape=out_shape,
    )(*args)


def _final_body(parts_ref, b_ref, out_ref):
    out_ref[...] = parts_ref[0] + parts_ref[1] + b_ref[...]


def _final(parts, b3):
    return pl.pallas_call(
        _final_body,
        grid=(GRID_M,),
        in_specs=[pl.BlockSpec((2, BM, D), lambda i: (0, i, 0)),
                  pl.BlockSpec((1, D), lambda i: (0, 0))],
        out_specs=pl.BlockSpec((BM, D), lambda i: (i, 0)),
        out_shape=jax.ShapeDtypeStruct((N, D), jnp.float32),
    )(parts, b3.reshape(1, D))


_GDN = lax.GatherDimensionNumbers(
    offset_dims=(), collapsed_slice_dims=(0,), start_index_map=(0,))


def _lane_splat(v, i):
    idx = jnp.zeros((L, 1), jnp.int32) + i
    return lax.gather(v, idx, _GDN, (1,),
                      mode=lax.GatherScatterMode.PROMISE_IN_BOUNDS)

def _spmm_body(col_split, p_hbm, src_hbm, dst_hbm, w_hbm, out_hbm,
               acc, srcb, dstb, wb, rows0, rows1,
               gsem0, gsem1, ssem0, ssem1):
    c = lax.axis_index("c")
    s = lax.axis_index("s")
    zero16 = jnp.zeros((16,), jnp.float32)
    rows = (rows0, rows1)
    gsem = (gsem0, gsem1)
    ssem = (ssem0, ssem1)

    def zrow(r, _):
        rr = rows0.at[r]
        for q in range(8):
            rr[pl.ds(q * 16, 16)] = zero16
        return 0
    lax.fori_loop(0, CH, zrow, 0)
    base = s * ROWS_PER_TILE
    for k in range(ROWS_PER_TILE // CH):
        pltpu.sync_copy(rows0, acc.at[pl.ds(base + k * CH, CH)])

    if col_split:
        groups = (2 * s, 2 * s + 1)
        table = p_hbm.at[c]
    else:
        groups = (c * NS + s,)
        table = p_hbm.at[0]

    for g in groups:
        for st in range(NST):
            pltpu.sync_copy(src_hbm.at[g, pl.ds(st * ST, ST)], srcb)
            pltpu.sync_copy(dst_hbm.at[g, pl.ds(st * ST, ST)], dstb)
            pltpu.sync_copy(w_hbm.at[g, pl.ds(st * ST, ST)], wb)

            def scale(buf, wrow):
                def srow(r, _):
                    rbase = (r // L) * L
                    wv16 = wrow[pl.ds(rbase, L)]
                    wsp = _lane_splat(wv16, r - rbase)
                    rr = buf.at[r]
                    for q in range(8):
                        sl = pl.ds(q * L, L)
                        rr[sl] = rr[sl] * wsp
                    return 0
                lax.fori_loop(0, CH, srow, 0)

            def chunk_pair(t, _):
                j0 = 2 * t
                j1 = 2 * t + 1
                g0 = pltpu.make_async_copy(table.at[srcb.at[j0]], rows0,
                                           gsem0)
                g0.start()
                g1 = pltpu.make_async_copy(table.at[srcb.at[j1]], rows1,
                                           gsem1)
                g1.start()
                g0.wait()
                scale(rows0, wb.at[j0])
                pltpu.async_copy(rows0, acc.at[dstb.at[j0]], ssem0, add=True)
                g1.wait()
                scale(rows1, wb.at[j1])
                pltpu.async_copy(rows1, acc.at[dstb.at[j1]], ssem1, add=True)
                pltpu.make_async_copy(rows0, acc.at[dstb.at[j0]],
                                      ssem0).wait()
                pltpu.make_async_copy(rows1, acc.at[dstb.at[j1]],
                                      ssem1).wait()
                return 0
            lax.fori_loop(0, ST // 2, chunk_pair, 0)

    plsc.subcore_barrier()
    pltpu.sync_copy(acc.at[pl.ds(base, ROWS_PER_TILE)],
                    out_hbm.at[c, pl.ds(base, ROWS_PER_TILE)])


def _make_spmm(col_split):
    mesh = plsc.VectorSubcoreMesh(core_axis_name="c", subcore_axis_name="s",
                                  num_cores=NC, num_subcores=NS)
    return pl.kernel(
        functools.partial(_spmm_body, col_split),
        out_type=jax.ShapeDtypeStruct((2, N_PAD, D), jnp.float32),
        mesh=mesh,
        scratch_types=[
            pltpu.VMEM_SHARED((N_PAD, D), jnp.float32),
            pltpu.VMEM((ST, CH), jnp.int32),
            pltpu.VMEM((ST, CH), jnp.int32),
            pltpu.VMEM((ST, CH), jnp.float32),
            pltpu.VMEM((CH, D), jnp.float32),
            pltpu.VMEM((CH, D), jnp.float32),
            pltpu.SemaphoreType.DMA,
            pltpu.SemaphoreType.DMA,
            pltpu.SemaphoreType.DMA,
            pltpu.SemaphoreType.DMA,
        ],
    )


def kernel(x, edge_index, edge_weight, W0, b0, W1, b1, W2, b2, W3, b3):
    E = edge_index.shape[1]
    pad = E_PAD - E
    src3 = jnp.pad(edge_index[0], (0, pad)).reshape(NG, NCH, CH)
    dst3 = jnp.pad(edge_index[1], (0, pad)).reshape(NG, NCH, CH)
    w3 = jnp.pad(edge_weight, (0, pad)).reshape(NG, NCH, CH)

    spmm256 = _make_spmm(True)
    spmm128 = _make_spmm(False)

    p = _mm0(x, W0)
    g0 = spmm256(p, src3, dst3, w3)
    h0, p1 = _act_mm(g0, b0.reshape(2, 1, D), None,
                     W1.reshape(2, 128, 256), True)
    g1 = spmm256(p1, src3, dst3, w3)
    h1, p2 = _act_mm(g1, b1.reshape(2, 1, D), h0,
                     W2.reshape(2, 128, 256), True)
    g2 = spmm256(p2, src3, dst3, w3)
    p3 = _act_mm(g2, b2.reshape(2, 1, D), h1,
                 W3.reshape(2, 128, 128), False)
    parts = spmm128(p3.reshape(1, N, D), src3, dst3, w3)
    return _final(parts, b3)


# speedup vs baseline: 1.2856x; 1.1275x over previous
import functools

import jax
import jax.numpy as jnp
from jax import lax
from jax.experimental import pallas as pl
from jax.experimental.pallas import tpu as pltpu
from jax.experimental.pallas import tpu_sc as plsc

N = 10000
N_PAD = 10240
D = 128
NC, NS, L = 2, 16, 16
BM = 1000
GRID_M = N // BM
CH = 64
NCH = 160
ST = 32
NST = NCH // ST
NG = NC * NS
E_PAD = NG * NCH * CH
ROWS_PER_TILE = N_PAD // NS


def _mm0_body(x_ref, w_ref, p_ref):
    p = jnp.dot(x_ref[...], w_ref[...], preferred_element_type=jnp.float32)
    p_ref[0] = p[:, :D]
    p_ref[1] = p[:, D:]


def _mm0(x, W0):
    return pl.pallas_call(
        _mm0_body,
        grid=(GRID_M,),
        in_specs=[pl.BlockSpec((BM, 128), lambda i: (i, 0)),
                  pl.BlockSpec((128, 256), lambda i: (0, 0))],
        out_specs=pl.BlockSpec((2, BM, D), lambda i: (0, i, 0)),
        out_shape=jax.ShapeDtypeStruct((2, N, D), jnp.float32),
    )(x, W0)


def _act_mm_body(has_prev, emit_h, g_ref, b_ref, *rest):
    if has_prev:
        prev_ref, w_ref = rest[0], rest[1]
        outs = rest[2:]
    else:
        w_ref = rest[0]
        outs = rest[1:]
    h0 = jax.nn.relu(g_ref[0] + b_ref[0])
    h1 = jax.nn.relu(g_ref[1] + b_ref[1])
    if has_prev:
        h0 = h0 + prev_ref[0]
        h1 = h1 + prev_ref[1]
    p = (jnp.dot(h0, w_ref[0], preferred_element_type=jnp.float32)
         + jnp.dot(h1, w_ref[1], preferred_element_type=jnp.float32))
    if emit_h:
        h_ref, p_ref = outs
        h_ref[0] = h0
        h_ref[1] = h1
        p_ref[0] = p[:, :D]
        p_ref[1] = p[:, D:]
    else:
        (p_ref,) = outs
        p_ref[...] = p


def _act_mm(g, b2, prev, Wr, emit_h):
    dout = Wr.shape[2]
    has_prev = prev is not None
    in_specs = [pl.BlockSpec((2, BM, D), lambda i: (0, i, 0)),
                pl.BlockSpec((2, 1, D), lambda i: (0, 0, 0))]
    args = [g, b2]
    if has_prev:
        in_specs.append(pl.BlockSpec((2, BM, D), lambda i: (0, i, 0)))
        args.append(prev)
    in_specs.append(pl.BlockSpec((2, 128, dout), lambda i: (0, 0, 0)))
    args.append(Wr)
    if emit_h:
        out_specs = [pl.BlockSpec((2, BM, D), lambda i: (0, i, 0)),
                     pl.BlockSpec((2, BM, D), lambda i: (0, i, 0))]
        out_shape = [jax.ShapeDtypeStruct((2, N, D), jnp.float32),
                     jax.ShapeDtypeStruct((2, N, D), jnp.float32)]
    else:
        out_specs = pl.BlockSpec((BM, dout), lambda i: (i, 0))
        out_shape = jax.ShapeDtypeStruct((N, dout), jnp.float32)
    return pl.pallas_call(
        functools.partial(_act_mm_body, has_prev, emit_h),
        grid=(GRID_M,),
        in_specs=in_specs,
        out_specs=out_specs,
        out_shape=out_shape,
    )(*args)


def _final_body(parts_ref, b_ref, out_ref):
    out_ref[...] = parts_ref[0] + parts_ref[1] + b_ref[...]


def _final(parts, b3):
    return pl.pallas_call(
        _final_body,
        grid=(GRID_M,),
        in_specs=[pl.BlockSpec((2, BM, D), lambda i: (0, i, 0)),
                  pl.BlockSpec((1, D), lambda i: (0, 0))],
        out_specs=pl.BlockSpec((BM, D), lambda i: (i, 0)),
        out_shape=jax.ShapeDtypeStruct((N, D), jnp.float32),
    )(parts, b3.reshape(1, D))


_GDN = lax.GatherDimensionNumbers(
    offset_dims=(), collapsed_slice_dims=(0,), start_index_map=(0,))


def _lane_splat(v, i):
    idx = jnp.zeros((L, 1), jnp.int32) + i
    return lax.gather(v, idx, _GDN, (1,),
                      mode=lax.GatherScatterMode.PROMISE_IN_BOUNDS)

def _spmm_body(col_split, p_hbm, src_hbm, dst_hbm, w_hbm, out_hbm,
               acc, srcb, dstb, wb, rows0, rows1,
               gsem0, gsem1, ssem0, ssem1):
    c = lax.axis_index("c")
    s = lax.axis_index("s")
    zero16 = jnp.zeros((16,), jnp.float32)
    rows = (rows0, rows1)
    gsem = (gsem0, gsem1)
    ssem = (ssem0, ssem1)

    def zrow(r, _):
        rr = rows0.at[r]
        for q in range(8):
            rr[pl.ds(q * 16, 16)] = zero16
        return 0
    lax.fori_loop(0, CH, zrow, 0)
    base = s * ROWS_PER_TILE
    for k in range(ROWS_PER_TILE // CH):
        pltpu.sync_copy(rows0, acc.at[pl.ds(base + k * CH, CH)])

    if col_split:
        groups = (2 * s, 2 * s + 1)
        table = p_hbm.at[c]
    else:
        groups = (c * NS + s,)
        table = p_hbm.at[0]

    for g in groups:
        for st in range(NST):
            pltpu.sync_copy(src_hbm.at[g, pl.ds(st * ST, ST)], srcb)
            pltpu.sync_copy(dst_hbm.at[g, pl.ds(st * ST, ST)], dstb)
            pltpu.sync_copy(w_hbm.at[g, pl.ds(st * ST, ST)], wb)

            def scale(buf, wrow):
                def sgrp(g, _):
                    rbase = g * L
                    wv16 = wrow[pl.ds(rbase, L)]
                    for i in range(L):
                        wsp = _lane_splat(wv16, i)
                        rr = buf.at[rbase + i]
                        for q in range(8):
                            sl = pl.ds(q * L, L)
                            rr[sl] = rr[sl] * wsp
                    return 0
                lax.fori_loop(0, CH // L, sgrp, 0)

            def chunk_pair(t, _):
                j0 = 2 * t
                j1 = 2 * t + 1
                g0 = pltpu.make_async_copy(table.at[srcb.at[j0]], rows0,
                                           gsem0)
                g0.start()
                g1 = pltpu.make_async_copy(table.at[srcb.at[j1]], rows1,
                                           gsem1)
                g1.start()
                g0.wait()
                scale(rows0, wb.at[j0])
                pltpu.async_copy(rows0, acc.at[dstb.at[j0]], ssem0, add=True)
                g1.wait()
                scale(rows1, wb.at[j1])
                pltpu.async_copy(rows1, acc.at[dstb.at[j1]], ssem1, add=True)
                pltpu.make_async_copy(rows0, acc.at[dstb.at[j0]],
                                      ssem0).wait()
                pltpu.make_async_copy(rows1, acc.at[dstb.at[j1]],
                                      ssem1).wait()
                return 0
            lax.fori_loop(0, ST // 2, chunk_pair, 0)

    plsc.subcore_barrier()
    pltpu.sync_copy(acc.at[pl.ds(base, ROWS_PER_TILE)],
                    out_hbm.at[c, pl.ds(base, ROWS_PER_TILE)])


def _make_spmm(col_split):
    mesh = plsc.VectorSubcoreMesh(core_axis_name="c", subcore_axis_name="s",
                                  num_cores=NC, num_subcores=NS)
    return pl.kernel(
        functools.partial(_spmm_body, col_split),
        out_type=jax.ShapeDtypeStruct((2, N_PAD, D), jnp.float32),
        mesh=mesh,
        scratch_types=[
            pltpu.VMEM_SHARED((N_PAD, D), jnp.float32),
            pltpu.VMEM((ST, CH), jnp.int32),
            pltpu.VMEM((ST, CH), jnp.int32),
            pltpu.VMEM((ST, CH), jnp.float32),
            pltpu.VMEM((CH, D), jnp.float32),
            pltpu.VMEM((CH, D), jnp.float32),
            pltpu.SemaphoreType.DMA,
            pltpu.SemaphoreType.DMA,
            pltpu.SemaphoreType.DMA,
            pltpu.SemaphoreType.DMA,
        ],
    )


def kernel(x, edge_index, edge_weight, W0, b0, W1, b1, W2, b2, W3, b3):
    E = edge_index.shape[1]
    pad = E_PAD - E
    src3 = jnp.pad(edge_index[0], (0, pad)).reshape(NG, NCH, CH)
    dst3 = jnp.pad(edge_index[1], (0, pad)).reshape(NG, NCH, CH)
    w3 = jnp.pad(edge_weight, (0, pad)).reshape(NG, NCH, CH)

    spmm256 = _make_spmm(True)
    spmm128 = _make_spmm(False)

    p = _mm0(x, W0)
    g0 = spmm256(p, src3, dst3, w3)
    h0, p1 = _act_mm(g0, b0.reshape(2, 1, D), None,
                     W1.reshape(2, 128, 256), True)
    g1 = spmm256(p1, src3, dst3, w3)
    h1, p2 = _act_mm(g1, b1.reshape(2, 1, D), h0,
                     W2.reshape(2, 128, 256), True)
    g2 = spmm256(p2, src3, dst3, w3)
    p3 = _act_mm(g2, b2.reshape(2, 1, D), h1,
                 W3.reshape(2, 128, 128), False)
    parts = spmm128(p3.reshape(1, N, D), src3, dst3, w3)
    return _final(parts, b3)
